# Initial kernel scaffold; baseline (speedup 1.0000x reference)
import jax
import jax.numpy as jnp
from jax.experimental import pallas as pl


def kernel(x, weights):
    raise NotImplementedError("write your pallas kernel here")


# speedup vs baseline: 3.3295x; 3.3295x over previous
import functools

import jax
import jax.numpy as jnp
from jax import lax
from jax.experimental import pallas as pl
from jax.experimental.pallas import tpu as pltpu
from jax.experimental.pallas import tpu_sc as plsc

_INFO = plsc.get_sparse_core_info()
_NC = _INFO.num_cores
_NS = _INFO.num_subcores
_NW = _NC * _NS

_CHUNK = 128


@functools.partial(jax.jit, static_argnames=("n_chunk", "d_model"))
def _gather(weights, idx3, n_chunk, d_model):
    total = _NW * n_chunk * _CHUNK
    per_w = n_chunk * _CHUNK
    mesh = plsc.VectorSubcoreMesh(core_axis_name="c", subcore_axis_name="s")

    @functools.partial(
        pl.kernel,
        out_type=jax.ShapeDtypeStruct((total, d_model), jnp.float32),
        mesh=mesh,
        scratch_types=[
            pltpu.VMEM((n_chunk, _CHUNK), jnp.int32),
            pltpu.VMEM((2, _CHUNK, d_model), jnp.float32),
            pltpu.SemaphoreType.DMA,
            pltpu.SemaphoreType.DMA,
        ],
    )
    def body(table_hbm, idx_hbm, out_hbm, idx_v, rows_v, sem0, sem1):
        wid = lax.axis_index("s") * _NC + lax.axis_index("c")
        base = wid * per_w
        pltpu.sync_copy(idx_hbm.at[wid], idx_v)
        pltpu.async_copy(table_hbm.at[idx_v.at[0]], rows_v.at[0], sem0)

        def step(t, carry):
            j0 = 2 * t
            j1 = j0 + 1
            pltpu.async_copy(table_hbm.at[idx_v.at[j1]], rows_v.at[1], sem1)
            pltpu.make_async_copy(
                table_hbm.at[idx_v.at[j0]], rows_v.at[0], sem0).wait()
            pltpu.sync_copy(
                rows_v.at[0], out_hbm.at[pl.ds(base + j0 * _CHUNK, _CHUNK)])
            @pl.when(t + 1 < n_chunk // 2)
            def _():
                pltpu.async_copy(
                    table_hbm.at[idx_v.at[j0 + 2]], rows_v.at[0], sem0)
            pltpu.make_async_copy(
                table_hbm.at[idx_v.at[j1]], rows_v.at[1], sem1).wait()
            pltpu.sync_copy(
                rows_v.at[1], out_hbm.at[pl.ds(base + j1 * _CHUNK, _CHUNK)])
            return carry

        lax.fori_loop(0, n_chunk // 2, step, 0)

    return body(weights, idx3)


def kernel(x, weights):
    b, s = x.shape
    d_model = weights.shape[1]
    total = b * s
    n_chunk = total // (_NW * _CHUNK)
    idx3 = x.reshape(_NW, n_chunk, _CHUNK)
    out = _gather(weights, idx3, n_chunk, d_model)
    return lax.stop_gradient(out.reshape(b, s, d_model))


# speedup vs baseline: 3.3365x; 1.0021x over previous
import functools

import jax
import jax.numpy as jnp
from jax import lax
from jax.experimental import pallas as pl
from jax.experimental.pallas import tpu as pltpu
from jax.experimental.pallas import tpu_sc as plsc

_INFO = plsc.get_sparse_core_info()
_NC = _INFO.num_cores
_NS = _INFO.num_subcores
_NW = _NC * _NS

_CHUNK = 128


@functools.partial(jax.jit, static_argnames=("n_chunk", "d_model"))
def _gather(weights, idx3, n_chunk, d_model):
    total = _NW * n_chunk * _CHUNK
    per_w = n_chunk * _CHUNK
    mesh = plsc.VectorSubcoreMesh(core_axis_name="c", subcore_axis_name="s")

    @functools.partial(
        pl.kernel,
        out_type=jax.ShapeDtypeStruct((total, d_model), jnp.float32),
        mesh=mesh,
        scratch_types=[
            pltpu.VMEM((n_chunk, _CHUNK), jnp.int32),
            pltpu.VMEM((4, _CHUNK, d_model), jnp.float32),
            [pltpu.SemaphoreType.DMA] * 4,
            [pltpu.SemaphoreType.DMA] * 4,
        ],
    )
    def body(table_hbm, idx_hbm, out_hbm, idx_v, rows_v, gsem, osem):
        wid = lax.axis_index("s") * _NC + lax.axis_index("c")
        base = wid * per_w
        pltpu.sync_copy(idx_hbm.at[wid], idx_v)

        def start_gather(j, b):
            pltpu.async_copy(table_hbm.at[idx_v.at[j]], rows_v.at[b], gsem[b])

        def wait_gather(j, b):
            pltpu.make_async_copy(
                table_hbm.at[idx_v.at[j]], rows_v.at[b], gsem[b]).wait()

        def start_out(j, b):
            pltpu.async_copy(
                rows_v.at[b],
                out_hbm.at[pl.ds(base + j * _CHUNK, _CHUNK)], osem[b])

        def wait_out(b):
            pltpu.make_async_copy(
                rows_v.at[b], out_hbm.at[pl.ds(base, _CHUNK)], osem[b]).wait()

        start_gather(0, 0)
        start_gather(1, 1)
        for j in (0, 1):
            wait_gather(j, j)
            start_out(j, j)
            start_gather(j + 2, j + 2)

        def step(t, carry):
            j0 = 4 * t + 2
            for k in range(4):
                j = j0 + k
                b = (2 + k) % 4
                bn = k
                wait_gather(j, b)
                start_out(j, b)
                wait_out(bn)
                start_gather(j + 2, bn)
            return carry

        lax.fori_loop(0, (n_chunk - 6) // 4, step, 0)

        for j in range(n_chunk - 4, n_chunk - 2):
            b = j % 4
            wait_gather(j, b)
            start_out(j, b)
            wait_out((j + 2) % 4)
            start_gather(j + 2, (j + 2) % 4)
        for j in range(n_chunk - 2, n_chunk):
            b = j % 4
            wait_gather(j, b)
            start_out(j, b)
        for b in range(4):
            wait_out(b)

    return body(weights, idx3)


def kernel(x, weights):
    b, s = x.shape
    d_model = weights.shape[1]
    total = b * s
    n_chunk = total // (_NW * _CHUNK)
    idx3 = x.reshape(_NW, n_chunk, _CHUNK)
    out = _gather(weights, idx3, n_chunk, d_model)
    return lax.stop_gradient(out.reshape(b, s, d_model))


# speedup vs baseline: 5.5029x; 1.6493x over previous
import functools

import jax
import jax.numpy as jnp
from jax import lax
from jax.experimental import pallas as pl
from jax.experimental.pallas import tpu as pltpu
from jax.experimental.pallas import tpu_sc as plsc

_INFO = plsc.get_sparse_core_info()
_NC = _INFO.num_cores
_NS = _INFO.num_subcores
_NW = _NC * _NS


@functools.partial(jax.jit, static_argnames=("b", "s", "d_model"))
def _gather(weights, x, b, s, d_model):
    per_w = b // _NW
    mesh = plsc.VectorSubcoreMesh(core_axis_name="c", subcore_axis_name="s")

    @functools.partial(
        pl.kernel,
        out_type=jax.ShapeDtypeStruct((b, s, d_model), jnp.float32),
        mesh=mesh,
        compiler_params=pltpu.CompilerParams(use_tc_tiling_on_sc=True),
        scratch_types=[
            pltpu.VMEM((per_w, s), jnp.int32),
            pltpu.VMEM((4, s, d_model), jnp.float32),
            [pltpu.SemaphoreType.DMA] * 4,
            [pltpu.SemaphoreType.DMA] * 4,
        ],
    )
    def body(table_hbm, idx_hbm, out_hbm, idx_v, rows_v, gsem, osem):
        wid = lax.axis_index("s") * _NC + lax.axis_index("c")
        base = wid * per_w
        pltpu.sync_copy(idx_hbm.at[pl.ds(base, per_w)], idx_v)

        def start_gather(j, bf):
            pltpu.async_copy(table_hbm.at[idx_v.at[j]], rows_v.at[bf], gsem[bf])

        def wait_gather(j, bf):
            pltpu.make_async_copy(
                table_hbm.at[idx_v.at[j]], rows_v.at[bf], gsem[bf]).wait()

        def start_out(j, bf):
            pltpu.async_copy(rows_v.at[bf], out_hbm.at[base + j], osem[bf])

        def wait_out(bf):
            pltpu.make_async_copy(
                rows_v.at[bf], out_hbm.at[base], osem[bf]).wait()

        start_gather(0, 0)
        start_gather(1, 1)
        for j in (0, 1):
            wait_gather(j, j)
            start_out(j, j)
            start_gather(j + 2, j + 2)

        def step(t, carry):
            j0 = 4 * t + 2
            for k in range(4):
                j = j0 + k
                bf = (2 + k) % 4
                bn = k
                wait_gather(j, bf)
                start_out(j, bf)
                wait_out(bn)
                start_gather(j + 2, bn)
            return carry

        n_loop = (per_w - 4) // 4
        lax.fori_loop(0, n_loop, step, 0)

        for j in range(4 * n_loop + 2, per_w - 2):
            bf = j % 4
            wait_gather(j, bf)
            start_out(j, bf)
            wait_out((j + 2) % 4)
            start_gather(j + 2, (j + 2) % 4)
        for j in range(per_w - 2, per_w):
            bf = j % 4
            wait_gather(j, bf)
            start_out(j, bf)
        for bf in range(4):
            wait_out(bf)

    return body(weights, x)


def kernel(x, weights):
    b, s = x.shape
    d_model = weights.shape[1]
    out = _gather(weights, x, b, s, d_model)
    return lax.stop_gradient(out)


# speedup vs baseline: 10.7423x; 1.9521x over previous
import functools

import jax
import jax.numpy as jnp
from jax import lax
from jax.experimental import pallas as pl
from jax.experimental.pallas import tpu as pltpu
from jax.experimental.pallas import tpu_sc as plsc

_INFO = plsc.get_sparse_core_info()
_NC = _INFO.num_cores
_NS = _INFO.num_subcores
_NW = _NC * _NS


@functools.partial(jax.jit, static_argnames=("b", "s", "d_model"))
def _gather(weights, xt, b, s, d_model):
    per_w = b // _NW
    mesh = plsc.VectorSubcoreMesh(core_axis_name="c", subcore_axis_name="s")

    @functools.partial(
        pl.kernel,
        out_type=jax.ShapeDtypeStruct((s, b, d_model), jnp.float32),
        mesh=mesh,
        compiler_params=pltpu.CompilerParams(use_tc_tiling_on_sc=True),
        scratch_types=[
            pltpu.VMEM((s, per_w), jnp.int32),
            pltpu.VMEM((4, per_w, d_model), jnp.float32),
            [pltpu.SemaphoreType.DMA] * 4,
            [pltpu.SemaphoreType.DMA] * 4,
        ],
    )
    def body(table_hbm, idx_hbm, out_hbm, idx_v, rows_v, gsem, osem):
        wid = lax.axis_index("s") * _NC + lax.axis_index("c")
        base = wid * per_w
        pltpu.sync_copy(idx_hbm.at[pl.ds(0, s), pl.ds(base, per_w)], idx_v)

        def start_gather(j, bf):
            pltpu.async_copy(table_hbm.at[idx_v.at[j]], rows_v.at[bf], gsem[bf])

        def wait_gather(j, bf):
            pltpu.make_async_copy(
                table_hbm.at[idx_v.at[j]], rows_v.at[bf], gsem[bf]).wait()

        def start_out(j, bf):
            pltpu.async_copy(
                rows_v.at[bf], out_hbm.at[j, pl.ds(base, per_w)], osem[bf])

        def wait_out(bf):
            pltpu.make_async_copy(
                rows_v.at[bf], out_hbm.at[0, pl.ds(base, per_w)],
                osem[bf]).wait()

        start_gather(0, 0)
        start_gather(1, 1)
        for j in (0, 1):
            wait_gather(j, j)
            start_out(j, j)
            start_gather(j + 2, j + 2)

        def step(t, carry):
            j0 = 4 * t + 2
            for k in range(4):
                j = j0 + k
                bf = (2 + k) % 4
                bn = k
                wait_gather(j, bf)
                start_out(j, bf)
                wait_out(bn)
                start_gather(j + 2, bn)
            return carry

        n_loop = (s - 4) // 4
        lax.fori_loop(0, n_loop, step, 0)

        for j in range(4 * n_loop + 2, s - 2):
            bf = j % 4
            wait_gather(j, bf)
            start_out(j, bf)
            wait_out((j + 2) % 4)
            start_gather(j + 2, (j + 2) % 4)
        for j in range(s - 2, s):
            bf = j % 4
            wait_gather(j, bf)
            start_out(j, bf)
        for bf in range(4):
            wait_out(bf)

    return body(weights, xt)


def kernel(x, weights):
    b, s = x.shape
    d_model = weights.shape[1]
    out = _gather(weights, x.T, b, s, d_model)
    return lax.stop_gradient(jnp.transpose(out, (1, 0, 2)))


# speedup vs baseline: 10.8232x; 1.0075x over previous
import functools

import jax
import jax.numpy as jnp
from jax import lax
from jax.experimental import pallas as pl
from jax.experimental.pallas import tpu as pltpu
from jax.experimental.pallas import tpu_sc as plsc

_INFO = plsc.get_sparse_core_info()
_NC = _INFO.num_cores
_NS = _INFO.num_subcores
_NW = _NC * _NS


@functools.partial(jax.jit, static_argnames=("b", "s", "d_model"))
def _gather(weights, xt, b, s, d_model):
    per_w = b // _NW
    mesh = plsc.VectorSubcoreMesh(core_axis_name="c", subcore_axis_name="s")

    @functools.partial(
        pl.kernel,
        out_type=jax.ShapeDtypeStruct((s, b, d_model), jnp.float32),
        mesh=mesh,
        compiler_params=pltpu.CompilerParams(use_tc_tiling_on_sc=True),
        scratch_types=[
            pltpu.VMEM((s, per_w), jnp.int32),
            pltpu.VMEM((6, per_w, d_model), jnp.float32),
            [pltpu.SemaphoreType.DMA] * 6,
            [pltpu.SemaphoreType.DMA] * 6,
        ],
    )
    def body(table_hbm, idx_hbm, out_hbm, idx_v, rows_v, gsem, osem):
        wid = lax.axis_index("s") * _NC + lax.axis_index("c")
        base = wid * per_w
        pltpu.sync_copy(idx_hbm.at[pl.ds(0, s), pl.ds(base, per_w)], idx_v)

        def start_gather(j, bf):
            pltpu.async_copy(table_hbm.at[idx_v.at[j]], rows_v.at[bf], gsem[bf])

        def wait_gather(j, bf):
            pltpu.make_async_copy(
                table_hbm.at[idx_v.at[j]], rows_v.at[bf], gsem[bf]).wait()

        def start_out(j, bf):
            pltpu.async_copy(
                rows_v.at[bf], out_hbm.at[j, pl.ds(base, per_w)], osem[bf])

        def wait_out(bf):
            pltpu.make_async_copy(
                rows_v.at[bf], out_hbm.at[0, pl.ds(base, per_w)],
                osem[bf]).wait()

        nbuf, ahead = 6, 3
        for g in range(ahead):
            start_gather(g, g)
        for j in range(ahead):
            wait_gather(j, j)
            start_out(j, j)
            start_gather(j + ahead, j + ahead)

        def step(t, carry):
            j0 = nbuf * t + ahead
            for k in range(nbuf):
                j = j0 + k
                bf = (ahead + k) % nbuf
                bn = k
                wait_gather(j, bf)
                start_out(j, bf)
                wait_out(bn)
                start_gather(j + ahead, bn)
            return carry

        n_loop = (s - 2 * ahead) // nbuf
        lax.fori_loop(0, n_loop, step, 0)

        for j in range(nbuf * n_loop + ahead, s - ahead):
            bf = j % nbuf
            wait_gather(j, bf)
            start_out(j, bf)
            wait_out((j + ahead) % nbuf)
            start_gather(j + ahead, (j + ahead) % nbuf)
        for j in range(s - ahead, s):
            bf = j % nbuf
            wait_gather(j, bf)
            start_out(j, bf)
        for bf in range(nbuf):
            wait_out(bf)

    return body(weights, xt)


def kernel(x, weights):
    b, s = x.shape
    d_model = weights.shape[1]
    out = _gather(weights, x.T, b, s, d_model)
    return lax.stop_gradient(jnp.transpose(out, (1, 0, 2)))


# speedup vs baseline: 10.8233x; 1.0000x over previous
import functools

import jax
import jax.numpy as jnp
from jax import lax
from jax.experimental import pallas as pl
from jax.experimental.pallas import tpu as pltpu
from jax.experimental.pallas import tpu_sc as plsc

_INFO = plsc.get_sparse_core_info()
_NC = _INFO.num_cores
_NS = _INFO.num_subcores
_NW = _NC * _NS


_SPLIT = 2
_NBUF = 12
_AHEAD = 6


@functools.partial(jax.jit, static_argnames=("b", "s", "d_model"))
def _gather(weights, xt, b, s, d_model):
    per_w = b // _NW
    ch = per_w // _SPLIT
    n_chunk = s * _SPLIT
    nbuf, ahead = _NBUF, _AHEAD
    mesh = plsc.VectorSubcoreMesh(core_axis_name="c", subcore_axis_name="s")

    @functools.partial(
        pl.kernel,
        out_type=jax.ShapeDtypeStruct((s, b, d_model), jnp.float32),
        mesh=mesh,
        compiler_params=pltpu.CompilerParams(use_tc_tiling_on_sc=True),
        scratch_types=[
            pltpu.VMEM((s, per_w), jnp.int32),
            pltpu.VMEM((nbuf, ch, d_model), jnp.float32),
            [pltpu.SemaphoreType.DMA] * nbuf,
            [pltpu.SemaphoreType.DMA] * nbuf,
        ],
    )
    def body(table_hbm, idx_hbm, out_hbm, idx_v, rows_v, gsem, osem):
        wid = lax.axis_index("s") * _NC + lax.axis_index("c")
        base = wid * per_w
        pltpu.sync_copy(idx_hbm.at[pl.ds(0, s), pl.ds(base, per_w)], idx_v)

        def start_gather(c, bf):
            j, h = c // _SPLIT, c % _SPLIT
            pltpu.async_copy(
                table_hbm.at[idx_v.at[j, pl.ds(h * ch, ch)]],
                rows_v.at[bf], gsem[bf])

        def wait_gather(c, bf):
            j, h = c // _SPLIT, c % _SPLIT
            pltpu.make_async_copy(
                table_hbm.at[idx_v.at[j, pl.ds(h * ch, ch)]],
                rows_v.at[bf], gsem[bf]).wait()

        def start_out(c, bf):
            j, h = c // _SPLIT, c % _SPLIT
            pltpu.async_copy(
                rows_v.at[bf],
                out_hbm.at[j, pl.ds(base + h * ch, ch)], osem[bf])

        def wait_out(bf):
            pltpu.make_async_copy(
                rows_v.at[bf], out_hbm.at[0, pl.ds(base, ch)], osem[bf]).wait()

        for g in range(ahead):
            start_gather(g, g)
        for c in range(ahead):
            wait_gather(c, c)
            start_out(c, c)
            start_gather(c + ahead, c + ahead)

        def step(t, carry):
            c0 = nbuf * t + ahead
            for k in range(nbuf):
                c = c0 + k
                bf = (ahead + k) % nbuf
                bn = (ahead + k + ahead) % nbuf
                wait_gather(c, bf)
                start_out(c, bf)
                wait_out(bn)
                start_gather(c + ahead, bn)
            return carry

        n_loop = (n_chunk - 2 * ahead) // nbuf
        lax.fori_loop(0, n_loop, step, 0)

        for c in range(nbuf * n_loop + ahead, n_chunk - ahead):
            bf = c % nbuf
            wait_gather(c, bf)
            start_out(c, bf)
            wait_out((c + ahead) % nbuf)
            start_gather(c + ahead, (c + ahead) % nbuf)
        for c in range(n_chunk - ahead, n_chunk):
            bf = c % nbuf
            wait_gather(c, bf)
            start_out(c, bf)
        for bf in range(nbuf):
            wait_out(bf)

    return body(weights, xt)


def kernel(x, weights):
    b, s = x.shape
    d_model = weights.shape[1]
    out = _gather(weights, x.T, b, s, d_model)
    return lax.stop_gradient(jnp.transpose(out, (1, 0, 2)))
